# baseline (speedup 1.0000x reference)
import jax
import jax.numpy as jnp
from jax.experimental import pallas as pl
from jax.experimental.pallas import tpu as pltpu

TAU = 0.8
LAMBDA = 0.5
EPS = 1e-8
LOG2E = 1.4426950408889634


def _tahin_kernel(zs_ref, zm_ref, w1_ref, b1_ref, w2_ref, b2_ref, pos_a_ref,
                  pos_b_ref, out_ref, zsh_ref, zmh_ref, acc_ref):
    i = pl.program_id(0)
    nb = pl.num_programs(0)
    half = pos_a_ref.shape[2]
    blk = 2 * half
    n = zs_ref.shape[0]

    @pl.when(i == 0)
    def _():
        w1 = w1_ref[...]
        b1 = b1_ref[...]
        w2 = w2_ref[...]
        b2 = b2_ref[...]

        def proj(x, scale):
            h = jnp.dot(x, w1, preferred_element_type=jnp.float32) + b1
            h = jnp.where(h > 0, h, jnp.exp(jnp.minimum(h, 0.0)) - 1.0)
            y = jnp.dot(h, w2, preferred_element_type=jnp.float32) + b2
            inv = scale * jax.lax.rsqrt(
                jnp.sum(y * y, axis=1, keepdims=True))
            return (y * inv).astype(jnp.bfloat16)

        zsh_ref[...] = proj(zs_ref[...], 1.0)
        zmh_ref[...] = proj(zm_ref[...], LOG2E / TAU)
        acc_ref[...] = jnp.zeros_like(acc_ref)

    dims = (((1,), (1,)), ((), ()))
    for h, pref in ((0, pos_a_ref), (1, pos_b_ref)):
        base = i * blk + h * half
        zsp = zsh_ref[pl.ds(base, half), :]
        zmp = zmh_ref[pl.ds(base, half), :]
        mask = pref[0, 0] != 0
        sim_r = jnp.exp2(jax.lax.dot_general(
            zsp, zmh_ref[...], dims, preferred_element_type=jnp.float32))
        r = jnp.sum(sim_r, axis=1)
        n1 = jnp.sum(jnp.where(mask, sim_r, 0.0), axis=1)
        sim_c = jnp.exp2(jax.lax.dot_general(
            zmp, zsh_ref[...], dims, preferred_element_type=jnp.float32))
        c = jnp.sum(sim_c, axis=1)
        n2 = jnp.sum(jnp.where(mask, sim_c, 0.0), axis=1)
        acc_ref[0, pl.ds(h * half, half)] += n1 / (r + EPS)
        acc_ref[1, pl.ds(h * half, half)] += n2 / (c + EPS)

    @pl.when(i == nb - 1)
    def _():
        loss_sc = -jnp.log(jnp.sum(acc_ref[0, :]) / n)
        loss_mp = -jnp.log(jnp.sum(acc_ref[1, :]) / n)
        loss = LAMBDA * loss_sc + (1.0 - LAMBDA) * loss_mp
        out_ref[...] = jnp.full((1, 1), loss, jnp.float32)


def kernel(z_sc, z_mp, pos, W1, b1, W2, b2):
    n, d = z_sc.shape
    blk = max(b for b in (400, 200, 80, 40, 16, 8) if n % b == 0)
    nb = n // blk
    half = blk // 2

    out = pl.pallas_call(
        _tahin_kernel,
        grid=(nb,),
        in_specs=[
            pl.BlockSpec((n, d), lambda i: (0, 0)),
            pl.BlockSpec((n, d), lambda i: (0, 0)),
            pl.BlockSpec((d, d), lambda i: (0, 0)),
            pl.BlockSpec((1, d), lambda i: (0, 0)),
            pl.BlockSpec((d, d), lambda i: (0, 0)),
            pl.BlockSpec((1, d), lambda i: (0, 0)),
            pl.BlockSpec((1, 1, half, n), lambda i: (i, 0, 0, 0)),
            pl.BlockSpec((1, 1, half, n), lambda i: (i, 1, 0, 0)),
        ],
        out_specs=pl.BlockSpec((1, 1), lambda i: (0, 0)),
        out_shape=jax.ShapeDtypeStruct((1, 1), jnp.float32),
        scratch_shapes=[
            pltpu.VMEM((n, d), jnp.bfloat16),
            pltpu.VMEM((n, d), jnp.bfloat16),
            pltpu.VMEM((2, blk), jnp.float32),
        ],
        compiler_params=pltpu.CompilerParams(
            dimension_semantics=("arbitrary",),
            vmem_limit_bytes=128 * 1024 * 1024),
    )(z_sc, z_mp, W1.T, b1.reshape(1, d), W2.T, b2.reshape(1, d),
      pos.reshape(nb, 2, half, n), pos.reshape(nb, 2, half, n))
    return out[0, 0]


# speedup vs baseline: 1.0378x; 1.0378x over previous
import jax
import jax.numpy as jnp
from jax.experimental import pallas as pl
from jax.experimental.pallas import tpu as pltpu

TAU = 0.8
LAMBDA = 0.5
EPS = 1e-8
LOG2E = 1.4426950408889634


def _tahin_kernel(zs_ref, zm_ref, w1_ref, b1_ref, w2_ref, b2_ref, pos_ref,
                  out_ref, zsh_ref, zmh_ref, acc_ref):
    i = pl.program_id(0)
    nb = pl.num_programs(0)
    blk = pos_ref.shape[1]
    n = zs_ref.shape[0]

    @pl.when(i == 0)
    def _():
        w1 = w1_ref[...]
        b1 = b1_ref[...]
        w2 = w2_ref[...]
        b2 = b2_ref[...]

        def proj(x, scale):
            h = jnp.dot(x, w1, preferred_element_type=jnp.float32) + b1
            h = jnp.where(h > 0, h, jnp.exp(jnp.minimum(h, 0.0)) - 1.0)
            y = jnp.dot(h, w2, preferred_element_type=jnp.float32) + b2
            inv = scale * jax.lax.rsqrt(
                jnp.sum(y * y, axis=1, keepdims=True))
            return (y * inv).astype(jnp.bfloat16)

        zsh_ref[...] = proj(zs_ref[...], 1.0)
        zmh_ref[...] = proj(zm_ref[...], LOG2E / TAU)
        acc_ref[...] = jnp.zeros_like(acc_ref)

    dims = (((1,), (1,)), ((), ()))
    zsp = zsh_ref[pl.ds(i * blk, blk), :]
    zmp = zmh_ref[pl.ds(i * blk, blk), :]
    mask = pos_ref[0] != 0
    sim_r = jnp.exp2(jax.lax.dot_general(
        zsp, zmh_ref[...], dims, preferred_element_type=jnp.float32))
    r = jnp.sum(sim_r, axis=1)
    n1 = jnp.sum(jnp.where(mask, sim_r, 0.0), axis=1)
    sim_c = jnp.exp2(jax.lax.dot_general(
        zmp, zsh_ref[...], dims, preferred_element_type=jnp.float32))
    c = jnp.sum(sim_c, axis=1)
    n2 = jnp.sum(jnp.where(mask, sim_c, 0.0), axis=1)
    acc_ref[0, :] += n1 / (r + EPS)
    acc_ref[1, :] += n2 / (c + EPS)

    @pl.when(i == nb - 1)
    def _():
        loss_sc = -jnp.log(jnp.sum(acc_ref[0, :]) / n)
        loss_mp = -jnp.log(jnp.sum(acc_ref[1, :]) / n)
        loss = LAMBDA * loss_sc + (1.0 - LAMBDA) * loss_mp
        out_ref[...] = jnp.full((1, 1), loss, jnp.float32)


def kernel(z_sc, z_mp, pos, W1, b1, W2, b2):
    n, d = z_sc.shape
    blk = max(b for b in (400, 200, 80, 40, 16, 8) if n % b == 0)
    nb = n // blk

    out = pl.pallas_call(
        _tahin_kernel,
        grid=(nb,),
        in_specs=[
            pl.BlockSpec((n, d), lambda i: (0, 0)),
            pl.BlockSpec((n, d), lambda i: (0, 0)),
            pl.BlockSpec((d, d), lambda i: (0, 0)),
            pl.BlockSpec((1, d), lambda i: (0, 0)),
            pl.BlockSpec((d, d), lambda i: (0, 0)),
            pl.BlockSpec((1, d), lambda i: (0, 0)),
            pl.BlockSpec((1, blk, n), lambda i: (i, 0, 0)),
        ],
        out_specs=pl.BlockSpec((1, 1), lambda i: (0, 0)),
        out_shape=jax.ShapeDtypeStruct((1, 1), jnp.float32),
        scratch_shapes=[
            pltpu.VMEM((n, d), jnp.bfloat16),
            pltpu.VMEM((n, d), jnp.bfloat16),
            pltpu.VMEM((2, blk), jnp.float32),
        ],
        compiler_params=pltpu.CompilerParams(
            dimension_semantics=("arbitrary",),
            vmem_limit_bytes=128 * 1024 * 1024),
    )(z_sc, z_mp, W1.T, b1.reshape(1, d), W2.T, b2.reshape(1, d),
      pos.reshape(nb, blk, n))
    return out[0, 0]
